# baseline (speedup 1.0000x reference)
import functools

import jax
import jax.numpy as jnp
from jax import lax
from jax.experimental import pallas as pl
from jax.experimental.pallas import tpu as pltpu
from jax.experimental.pallas import tpu_sc as plsc

_N_E = 1024
_E_DIM = 64
_BETA = 0.25
_N_ROWS = 32 * 32 * 32
_BM = 2048

_NC = 2
_NS = 16
_NW = _NC * _NS
_ROWS_PER_W = _N_ROWS // _NW
_CHUNK = 128
_NCHUNK = _ROWS_PER_W // _CHUNK
_GROUP = 256
_NGROUP = _ROWS_PER_W // _GROUP
_CPG = _GROUP // _CHUNK
_PAD_DIM = 128
_REPL = 8


def _vq_body(z_ref, emb_ref, lane_ref, oh_ref, idx_ref, idx_sc_ref, loss_ref):
    i = pl.program_id(0)
    z = z_ref[...]
    emb = emb_ref[...]
    z2 = jnp.sum(z * z, axis=1, keepdims=True)
    e2 = jnp.sum(emb * emb, axis=1)
    s = lax.dot_general(z, emb, (((1,), (1,)), ((), ())),
                        preferred_element_type=jnp.float32)
    d = (z2 + e2[None, :]) - 2.0 * s
    dmin = jnp.min(d, axis=1, keepdims=True)
    lane = lane_ref[...]
    idx_f = jnp.min(jnp.where(d == dmin, lane, float(_N_E)), axis=1,
                    keepdims=True)
    oh_ref[...] = (lane == idx_f).astype(jnp.float32)
    idx = idx_f.astype(jnp.int32)
    idx_ref[...] = idx
    row = lax.broadcasted_iota(jnp.int32, (_BM, 1), 0) + i * _BM
    rep = (row // _CHUNK) % _REPL
    idx_sc_ref[...] = idx + rep * _N_E
    part = jnp.sum(dmin).reshape(1, 1)

    @pl.when(i == 0)
    def _():
        loss_ref[...] = jnp.zeros((1, 1), jnp.float32)

    loss_ref[...] += part


@functools.partial(
    pl.kernel,
    out_type=jax.ShapeDtypeStruct((_N_ROWS, _PAD_DIM), jnp.float32),
    scratch_types=[
        pltpu.VMEM((_NCHUNK, _CHUNK), jnp.int32),
        pltpu.VMEM((_GROUP, _PAD_DIM), jnp.float32),
        pltpu.VMEM((_GROUP, _PAD_DIM), jnp.float32),
        pltpu.SemaphoreType.DMA,
        pltpu.SemaphoreType.DMA,
        pltpu.SemaphoreType.DMA,
        pltpu.SemaphoreType.DMA,
    ],
    mesh=plsc.VectorSubcoreMesh(core_axis_name="c", subcore_axis_name="s"),
)
def _gather_rows(emb_hbm, idx_hbm, out_hbm, idx_v, rows0, rows1,
                 sg0, sg1, so0, so1):
    wid = lax.axis_index("s") * _NC + lax.axis_index("c")
    base = wid * _ROWS_PER_W
    pltpu.sync_copy(idx_hbm.at[pl.ds(wid * _NCHUNK, _NCHUNK)], idx_v)
    rows = (rows0, rows1)
    sg = (sg0, sg1)
    so = (so0, so1)

    def fire(g):
        b = g & 1
        return [
            pltpu.async_copy(emb_hbm.at[idx_v.at[g * _CPG + j]],
                             rows[b].at[pl.ds(j * _CHUNK, _CHUNK)], sg[b])
            for j in range(_CPG)
        ]

    gathers = fire(0)
    outs = [None, None]
    for g in range(_NGROUP):
        b = g & 1
        if g + 1 < _NGROUP:
            nb = (g + 1) & 1
            if outs[nb] is not None:
                outs[nb].wait()
            nxt = fire(g + 1)
        for c in gathers:
            c.wait()
        outs[b] = pltpu.async_copy(
            rows[b], out_hbm.at[pl.ds(base + g * _GROUP, _GROUP)], so[b])
        if g + 1 < _NGROUP:
            gathers = nxt
    for o in outs:
        o.wait()


def kernel(z, emb):
    zp = jnp.transpose(z, (0, 2, 3, 1))
    z_flat = zp.reshape(-1, _E_DIM)
    grid = _N_ROWS // _BM
    oh, idx, idx_sc, loss_sum = pl.pallas_call(
        _vq_body,
        grid=(grid,),
        in_specs=[
            pl.BlockSpec((_BM, _E_DIM), lambda i: (i, 0)),
            pl.BlockSpec((_N_E, _E_DIM), lambda i: (0, 0)),
            pl.BlockSpec((1, _N_E), lambda i: (0, 0)),
        ],
        out_specs=[
            pl.BlockSpec((_BM, _N_E), lambda i: (i, 0)),
            pl.BlockSpec((_BM, 1), lambda i: (i, 0)),
            pl.BlockSpec((_BM, 1), lambda i: (i, 0)),
            pl.BlockSpec((1, 1), lambda i: (0, 0)),
        ],
        out_shape=[
            jax.ShapeDtypeStruct((_N_ROWS, _N_E), jnp.float32),
            jax.ShapeDtypeStruct((_N_ROWS, 1), jnp.int32),
            jax.ShapeDtypeStruct((_N_ROWS, 1), jnp.int32),
            jax.ShapeDtypeStruct((1, 1), jnp.float32),
        ],
    )(z_flat, emb, lax.iota(jnp.float32, _N_E).reshape(1, _N_E))
    emb_pad = jnp.pad(emb, ((0, 0), (0, _PAD_DIM - _E_DIM)))
    emb_repl = jnp.tile(emb_pad, (_REPL, 1))
    zq_pad = _gather_rows(emb_repl, idx_sc.reshape(_NW * _NCHUNK, _CHUNK))
    zq_flat = zq_pad[:, :_E_DIM]
    loss = (1.0 + _BETA) * loss_sum[0, 0] / (_N_ROWS * _E_DIM)
    z_q = jnp.transpose(zq_flat.reshape(zp.shape), (0, 3, 1, 2))
    return (loss, z_q, oh, idx)


# speedup vs baseline: 1.1269x; 1.1269x over previous
import functools

import jax
import jax.numpy as jnp
from jax import lax
from jax.experimental import pallas as pl
from jax.experimental.pallas import tpu as pltpu
from jax.experimental.pallas import tpu_sc as plsc

_N_E = 1024
_E_DIM = 64
_BETA = 0.25
_N_ROWS = 32 * 32 * 32
_BM = 2048

_NC = 2
_NS = 16
_NW = _NC * _NS
_ROWS_PER_W = _N_ROWS // _NW
_CHUNK = 128
_NCHUNK = _ROWS_PER_W // _CHUNK
_GROUP = 256
_NGROUP = _ROWS_PER_W // _GROUP
_CPG = _GROUP // _CHUNK
_PAD_DIM = 128
_REPL = 8


def _vq_body(z_ref, emb_ref, lane_ref, oh_ref, idx_ref, loss_ref):
    i = pl.program_id(0)
    z = z_ref[...]
    emb = emb_ref[...]
    z2 = jnp.sum(z * z, axis=1, keepdims=True)
    e2 = jnp.sum(emb * emb, axis=1)
    s = lax.dot_general(z, emb, (((1,), (1,)), ((), ())),
                        preferred_element_type=jnp.float32)
    d = (z2 + e2[None, :]) - 2.0 * s
    dmin = jnp.min(d, axis=1, keepdims=True)
    lane = lane_ref[...]
    idx_f = jnp.min(jnp.where(d == dmin, lane, float(_N_E)), axis=1,
                    keepdims=True)
    oh_ref[...] = (lane == idx_f).astype(jnp.float32)
    idx_ref[...] = idx_f.astype(jnp.int32)
    part = jnp.sum(dmin).reshape(1, 1)

    @pl.when(i == 0)
    def _():
        loss_ref[...] = jnp.zeros((1, 1), jnp.float32)

    loss_ref[...] += part


@functools.partial(
    pl.kernel,
    out_type=jax.ShapeDtypeStruct((_N_ROWS, _PAD_DIM), jnp.float32),
    scratch_types=[
        pltpu.VMEM((_NCHUNK, _CHUNK), jnp.int32),
        pltpu.VMEM((_GROUP, _PAD_DIM), jnp.float32),
        pltpu.VMEM((_GROUP, _PAD_DIM), jnp.float32),
        pltpu.SemaphoreType.DMA,
        pltpu.SemaphoreType.DMA,
        pltpu.SemaphoreType.DMA,
        pltpu.SemaphoreType.DMA,
    ],
    mesh=plsc.VectorSubcoreMesh(core_axis_name="c", subcore_axis_name="s"),
)
def _gather_rows(emb_hbm, idx_hbm, out_hbm, idx_v, rows0, rows1,
                 sg0, sg1, so0, so1):
    wid = lax.axis_index("s") * _NC + lax.axis_index("c")
    base = wid * _ROWS_PER_W
    pltpu.sync_copy(idx_hbm.at[pl.ds(wid * _NCHUNK, _NCHUNK)], idx_v)
    rows = (rows0, rows1)
    sg = (sg0, sg1)
    so = (so0, so1)

    def fire(g):
        b = g & 1
        return [
            pltpu.async_copy(emb_hbm.at[idx_v.at[g * _CPG + j]],
                             rows[b].at[pl.ds(j * _CHUNK, _CHUNK)], sg[b])
            for j in range(_CPG)
        ]

    gathers = fire(0)
    outs = [None, None]
    for g in range(_NGROUP):
        b = g & 1
        if g + 1 < _NGROUP:
            nb = (g + 1) & 1
            if outs[nb] is not None:
                outs[nb].wait()
            nxt = fire(g + 1)
        for c in gathers:
            c.wait()
        outs[b] = pltpu.async_copy(
            rows[b], out_hbm.at[pl.ds(base + g * _GROUP, _GROUP)], so[b])
        if g + 1 < _NGROUP:
            gathers = nxt
    for o in outs:
        o.wait()


def kernel(z, emb):
    zp = jnp.transpose(z, (0, 2, 3, 1))
    z_flat = zp.reshape(-1, _E_DIM)
    grid = _N_ROWS // _BM
    oh, idx, loss_sum = pl.pallas_call(
        _vq_body,
        grid=(grid,),
        in_specs=[
            pl.BlockSpec((_BM, _E_DIM), lambda i: (i, 0)),
            pl.BlockSpec((_N_E, _E_DIM), lambda i: (0, 0)),
            pl.BlockSpec((1, _N_E), lambda i: (0, 0)),
        ],
        out_specs=[
            pl.BlockSpec((_BM, _N_E), lambda i: (i, 0)),
            pl.BlockSpec((_BM, 1), lambda i: (i, 0)),
            pl.BlockSpec((1, 1), lambda i: (0, 0)),
        ],
        out_shape=[
            jax.ShapeDtypeStruct((_N_ROWS, _N_E), jnp.float32),
            jax.ShapeDtypeStruct((_N_ROWS, 1), jnp.int32),
            jax.ShapeDtypeStruct((1, 1), jnp.float32),
        ],
    )(z_flat, emb, lax.iota(jnp.float32, _N_E).reshape(1, _N_E))
    emb_pad = jnp.pad(emb, ((0, 0), (0, _PAD_DIM - _E_DIM)))
    emb_repl = jnp.tile(emb_pad, (_REPL, 1))
    idx_grid = idx.reshape(_NW * _NCHUNK, _CHUNK)
    repl_off = (jnp.arange(_NW * _NCHUNK, dtype=jnp.int32) % _REPL) * _N_E
    zq_pad = _gather_rows(emb_repl, idx_grid + repl_off[:, None])
    zq_flat = zq_pad[:, :_E_DIM]
    loss = (1.0 + _BETA) * loss_sum[0, 0] / (_N_ROWS * _E_DIM)
    z_q = jnp.transpose(zq_flat.reshape(zp.shape), (0, 3, 1, 2))
    return (loss, z_q, oh, idx)


# speedup vs baseline: 1.1333x; 1.0057x over previous
import functools

import jax
import jax.numpy as jnp
from jax import lax
from jax.experimental import pallas as pl
from jax.experimental.pallas import tpu as pltpu
from jax.experimental.pallas import tpu_sc as plsc

_N_E = 1024
_E_DIM = 64
_BETA = 0.25
_N_ROWS = 32 * 32 * 32
_BM = 2048

_NC = 2
_NS = 16
_NW = _NC * _NS
_ROWS_PER_W = _N_ROWS // _NW
_CHUNK = 64
_NCHUNK = _ROWS_PER_W // _CHUNK
_GROUP = 256
_NGROUP = _ROWS_PER_W // _GROUP
_CPG = _GROUP // _CHUNK
_PAD_DIM = 128
_REPL = 8


def _vq_body(z_ref, emb_ref, lane_ref, oh_ref, idx_ref, loss_ref):
    i = pl.program_id(0)
    z = z_ref[...]
    emb = emb_ref[...]
    z2 = jnp.sum(z * z, axis=1, keepdims=True)
    e2 = jnp.sum(emb * emb, axis=1)
    s = lax.dot_general(z, emb, (((1,), (1,)), ((), ())),
                        preferred_element_type=jnp.float32)
    d = (z2 + e2[None, :]) - 2.0 * s
    dmin = jnp.min(d, axis=1, keepdims=True)
    lane = lane_ref[...]
    idx_f = jnp.min(jnp.where(d == dmin, lane, float(_N_E)), axis=1,
                    keepdims=True)
    oh_ref[...] = (lane == idx_f).astype(jnp.float32)
    idx_ref[...] = idx_f.astype(jnp.int32)
    part = jnp.sum(dmin).reshape(1, 1)

    @pl.when(i == 0)
    def _():
        loss_ref[...] = jnp.zeros((1, 1), jnp.float32)

    loss_ref[...] += part


@functools.partial(
    pl.kernel,
    out_type=jax.ShapeDtypeStruct((_N_ROWS, _PAD_DIM), jnp.float32),
    scratch_types=[
        pltpu.VMEM((_NCHUNK, _CHUNK), jnp.int32),
        pltpu.VMEM((_GROUP, _PAD_DIM), jnp.float32),
        pltpu.VMEM((_GROUP, _PAD_DIM), jnp.float32),
        pltpu.SemaphoreType.DMA,
        pltpu.SemaphoreType.DMA,
        pltpu.SemaphoreType.DMA,
        pltpu.SemaphoreType.DMA,
    ],
    mesh=plsc.VectorSubcoreMesh(core_axis_name="c", subcore_axis_name="s"),
)
def _gather_rows(emb_hbm, idx_hbm, out_hbm, idx_v, rows0, rows1,
                 sg0, sg1, so0, so1):
    wid = lax.axis_index("s") * _NC + lax.axis_index("c")
    base = wid * _ROWS_PER_W
    pltpu.sync_copy(idx_hbm.at[pl.ds(wid * _NCHUNK, _NCHUNK)], idx_v)
    rows = (rows0, rows1)
    sg = (sg0, sg1)
    so = (so0, so1)

    def fire(g):
        b = g & 1
        return [
            pltpu.async_copy(emb_hbm.at[idx_v.at[g * _CPG + j]],
                             rows[b].at[pl.ds(j * _CHUNK, _CHUNK)], sg[b])
            for j in range(_CPG)
        ]

    gathers = fire(0)
    outs = [None, None]
    for g in range(_NGROUP):
        b = g & 1
        if g + 1 < _NGROUP:
            nb = (g + 1) & 1
            if outs[nb] is not None:
                outs[nb].wait()
            nxt = fire(g + 1)
        for c in gathers:
            c.wait()
        outs[b] = pltpu.async_copy(
            rows[b], out_hbm.at[pl.ds(base + g * _GROUP, _GROUP)], so[b])
        if g + 1 < _NGROUP:
            gathers = nxt
    for o in outs:
        o.wait()


def kernel(z, emb):
    zp = jnp.transpose(z, (0, 2, 3, 1))
    z_flat = zp.reshape(-1, _E_DIM)
    grid = _N_ROWS // _BM
    oh, idx, loss_sum = pl.pallas_call(
        _vq_body,
        grid=(grid,),
        in_specs=[
            pl.BlockSpec((_BM, _E_DIM), lambda i: (i, 0)),
            pl.BlockSpec((_N_E, _E_DIM), lambda i: (0, 0)),
            pl.BlockSpec((1, _N_E), lambda i: (0, 0)),
        ],
        out_specs=[
            pl.BlockSpec((_BM, _N_E), lambda i: (i, 0)),
            pl.BlockSpec((_BM, 1), lambda i: (i, 0)),
            pl.BlockSpec((1, 1), lambda i: (0, 0)),
        ],
        out_shape=[
            jax.ShapeDtypeStruct((_N_ROWS, _N_E), jnp.float32),
            jax.ShapeDtypeStruct((_N_ROWS, 1), jnp.int32),
            jax.ShapeDtypeStruct((1, 1), jnp.float32),
        ],
    )(z_flat, emb, lax.iota(jnp.float32, _N_E).reshape(1, _N_E))
    emb_pad = jnp.pad(emb, ((0, 0), (0, _PAD_DIM - _E_DIM)))
    emb_repl = jnp.tile(emb_pad, (_REPL, 1))
    idx_grid = idx.reshape(_NW * _NCHUNK, _CHUNK)
    repl_off = (jnp.arange(_NW * _NCHUNK, dtype=jnp.int32) % _REPL) * _N_E
    zq_pad = _gather_rows(emb_repl, idx_grid + repl_off[:, None])
    zq_flat = zq_pad[:, :_E_DIM]
    loss = (1.0 + _BETA) * loss_sum[0, 0] / (_N_ROWS * _E_DIM)
    z_q = jnp.transpose(zq_flat.reshape(zp.shape), (0, 3, 1, 2))
    return (loss, z_q, oh, idx)
